# baseline (speedup 1.0000x reference)
import jax
import jax.numpy as jnp
from jax.experimental import pallas as pl
from jax.experimental.pallas import tpu as pltpu

_TB = 256


def _vq_kernel(xt_ref, cb_ref, idx_ref, q_ref, c2_ref):
    xt = xt_ref[...]
    cb = cb_ref[...]
    @pl.when(pl.program_id(0) == 0)
    def _():
        c2_ref[...] = jnp.sum(cb ** 2, axis=-1)
    mm = jax.lax.dot_general(xt, cb, (((1,), (1,)), ((), ())),
                             preferred_element_type=jnp.float32)
    x2 = jnp.sum(xt ** 2, axis=-1, keepdims=True)
    c2 = c2_ref[...]
    dist = x2 + c2[None, :] - 2.0 * mm
    idx = jnp.argmin(dist, axis=1)
    idx_ref[...] = idx
    onehot = (jax.lax.broadcasted_iota(jnp.int32, dist.shape, 1)
              == idx[:, None]).astype(jnp.float32)
    q_ref[...] = jax.lax.dot_general(onehot, cb, (((1,), (0,)), ((), ())),
                                     preferred_element_type=jnp.float32)


def kernel(x, codebook):
    B, D, T = x.shape
    K = codebook.shape[0]
    xt = jnp.transpose(x, (0, 2, 1)).reshape(B * T, D)
    n_blocks = (B * T) // _TB
    idx, q = pl.pallas_call(
        _vq_kernel,
        grid=(n_blocks,),
        in_specs=[pl.BlockSpec((_TB, D), lambda i: (i, 0)),
                  pl.BlockSpec((K, D), lambda i: (0, 0))],
        out_specs=[pl.BlockSpec((_TB,), lambda i: (i,)),
                   pl.BlockSpec((_TB, D), lambda i: (i, 0))],
        out_shape=[jax.ShapeDtypeStruct((B * T,), jnp.int32),
                   jax.ShapeDtypeStruct((B * T, D), jnp.float32)],
        scratch_shapes=[pltpu.VMEM((K,), jnp.float32)],
    )(xt, codebook)
    quantized = jnp.transpose(q.reshape(B, T, D), (0, 2, 1))
    return quantized, idx.reshape(B, T)


# speedup vs baseline: 1.0264x; 1.0264x over previous
import jax
import jax.numpy as jnp
from jax.experimental import pallas as pl


def _vq_kernel(x_ref, cb_ref, idx_ref, q_ref):
    xs = x_ref[0]
    cb = cb_ref[...]
    xt = xs.T
    mm = jax.lax.dot_general(xt, cb, (((1,), (1,)), ((), ())),
                             preferred_element_type=jnp.float32)
    x2 = jnp.sum(xt ** 2, axis=-1, keepdims=True)
    c2 = jnp.sum(cb ** 2, axis=-1)
    dist = x2 + c2[None, :] - 2.0 * mm
    idx = jnp.argmin(dist, axis=1)
    idx_ref[0, 0] = idx
    onehot = (jax.lax.broadcasted_iota(jnp.int32, dist.shape, 1)
              == idx[:, None]).astype(jnp.float32)
    q_ref[0] = jax.lax.dot_general(cb, onehot, (((0,), (1,)), ((), ())),
                                   preferred_element_type=jnp.float32)


def kernel(x, codebook):
    B, D, T = x.shape
    K = codebook.shape[0]
    q, idx = pl.pallas_call(
        _vq_kernel,
        grid=(B,),
        in_specs=[pl.BlockSpec((1, D, T), lambda b: (b, 0, 0)),
                  pl.BlockSpec((K, D), lambda b: (0, 0))],
        out_specs=[pl.BlockSpec((1, 1, T), lambda b: (b, 0, 0)),
                   pl.BlockSpec((1, D, T), lambda b: (b, 0, 0))],
        out_shape=[jax.ShapeDtypeStruct((B, 1, T), jnp.int32),
                   jax.ShapeDtypeStruct((B, D, T), jnp.float32)],
    )(x, codebook)[::-1]
    return q, idx.reshape(B, T)


# speedup vs baseline: 1.0387x; 1.0120x over previous
import functools

import jax
import jax.numpy as jnp
from jax import lax
from jax.experimental import pallas as pl
from jax.experimental.pallas import tpu as pltpu
from jax.experimental.pallas import tpu_sc as plsc


def _vq_tc_kernel(x_ref, cb_ref, idx_ref):
    xs = x_ref[0]
    cb = cb_ref[...]
    xt = xs.T
    mm = jax.lax.dot_general(xt, cb, (((1,), (1,)), ((), ())),
                             preferred_element_type=jnp.float32)
    x2 = jnp.sum(xt ** 2, axis=-1, keepdims=True)
    c2 = jnp.sum(cb ** 2, axis=-1)
    dist = x2 + c2[None, :] - 2.0 * mm
    idx_ref[0, 0] = jnp.argmin(dist, axis=1)


def _make_sc_gather(n_rows, d, n_workers, nc):
    rows_per_w = n_rows // n_workers

    @functools.partial(
        pl.kernel,
        mesh=plsc.VectorSubcoreMesh(core_axis_name="c", subcore_axis_name="s"),
        out_type=jax.ShapeDtypeStruct((n_rows, d), jnp.float32),
        scratch_types=[
            pltpu.VMEM((rows_per_w,), jnp.int32),
            pltpu.VMEM((rows_per_w, d), jnp.float32),
            pltpu.SemaphoreType.DMA,
        ],
    )
    def sc_gather(table_hbm, idx_hbm, out_hbm, idx_v, rows_v, sem):
        wid = lax.axis_index("s") * nc + lax.axis_index("c")
        base = wid * rows_per_w
        pltpu.sync_copy(idx_hbm.at[pl.ds(base, rows_per_w)], idx_v)
        pltpu.async_copy(table_hbm.at[idx_v], rows_v, sem).wait()
        pltpu.sync_copy(rows_v, out_hbm.at[pl.ds(base, rows_per_w)])

    return sc_gather


def kernel(x, codebook):
    B, D, T = x.shape
    K = codebook.shape[0]
    idx3 = pl.pallas_call(
        _vq_tc_kernel,
        grid=(B,),
        in_specs=[pl.BlockSpec((1, D, T), lambda b: (b, 0, 0)),
                  pl.BlockSpec((K, D), lambda b: (0, 0))],
        out_specs=pl.BlockSpec((1, 1, T), lambda b: (b, 0, 0)),
        out_shape=jax.ShapeDtypeStruct((B, 1, T), jnp.int32),
    )(x, codebook)
    idx_flat = idx3.reshape(B * T)
    info = plsc.get_sparse_core_info()
    nw = info.num_cores * info.num_subcores
    rows = _make_sc_gather(B * T, D, nw, info.num_cores)(codebook, idx_flat)
    quantized = jnp.transpose(rows.reshape(B, T, D), (0, 2, 1))
    return quantized, idx3.reshape(B, T)
